# Initial kernel scaffold; baseline (speedup 1.0000x reference)
import jax
import jax.numpy as jnp
from jax.experimental import pallas as pl


def kernel(x, edge_index, neg_idx, batch, eps, W_in, W_msg, W_mu, W_logstd, W_dec, b_dec):
    raise NotImplementedError("write your pallas kernel here")


# speedup vs baseline: 2.5617x; 2.5617x over previous
import functools

import jax
import jax.numpy as jnp
from jax import lax
from jax.experimental import pallas as pl
from jax.experimental.pallas import tpu as pltpu
from jax.experimental.pallas import tpu_sc as plsc

N = 10000
E = 320000
D = 128
S = 64
TWO_E = 2 * E

NC, NS, L = 2, 16, 16
NW = NC * NS
NPAD = 10240
C = 128
EPW = 79 * C
EPAD = EPW * NW
EPW2 = 157 * C
EPAD2 = EPW2 * NW

BLK = 1024
GRID = NPAD // BLK
FROWS = EPAD2 // C
FGRID = 4
FBLK = FROWS // FGRID
POS_ROWS = E // C
VALID_ROWS = TWO_E // C


def _sc_segment_sum_body(hm_hbm, srcp_hbm, dstp_hbm, zeros_hbm, out_hbm,
                         src_v, dst_v, rows_v, acc_sh, sem):
    cid = lax.axis_index("c")
    sid = lax.axis_index("s")
    wid = sid * NC + cid
    zrows = NPAD // NS
    pltpu.sync_copy(zeros_hbm, acc_sh.at[pl.ds(sid * zrows, zrows)])
    plsc.subcore_barrier()

    def body(j, carry):
        base = wid * EPW + j * C
        pltpu.sync_copy(srcp_hbm.at[pl.ds(base, C)], src_v)
        pltpu.sync_copy(dstp_hbm.at[pl.ds(base, C)], dst_v)
        pltpu.async_copy(hm_hbm.at[src_v], rows_v, sem).wait()
        pltpu.sync_copy(rows_v, acc_sh.at[dst_v], add=True)
        return carry

    lax.fori_loop(0, EPW // C, body, 0)
    plsc.subcore_barrier()
    rpw = NPAD // NS
    pltpu.sync_copy(acc_sh.at[pl.ds(sid * rpw, rpw)],
                    out_hbm.at[cid, pl.ds(sid * rpw, rpw)])


def _sc_edge_dot_body(z_hbm, zw2_hbm, u_hbm, v_hbm, out_hbm,
                      u_v, v_v, ub, vb, ev, sem1, sem2):
    cid = lax.axis_index("c")
    sid = lax.axis_index("s")
    wid = sid * NC + cid
    iota = lax.iota(jnp.int32, L)
    rows = [iota + g * L for g in range(C // L)]
    zero16 = jnp.zeros((L,), jnp.float32)

    def body(j, carry):
        base = wid * EPW2 + j * C
        pltpu.sync_copy(u_hbm.at[pl.ds(base, C)], u_v)
        pltpu.sync_copy(v_hbm.at[pl.ds(base, C)], v_v)
        cp1 = pltpu.async_copy(z_hbm.at[u_v], ub, sem1)
        cp2 = pltpu.async_copy(zw2_hbm.at[v_v], vb, sem2)
        cp1.wait()
        cp2.wait()

        def sbody(s, accs):
            col = jnp.full((L,), 0, jnp.int32) + s
            return tuple(
                accs[g]
                + plsc.load_gather(ub, [rows[g], col])
                * plsc.load_gather(vb, [rows[g], col])
                for g in range(C // L))

        accs = lax.fori_loop(0, S, sbody, tuple(zero16 for _ in range(C // L)))
        for g in range(C // L):
            ev[pl.ds(g * L, L)] = accs[g]
        pltpu.sync_copy(ev, out_hbm.at[pl.ds(base, C)])
        return carry

    lax.fori_loop(0, EPW2 // C, body, 0)


@functools.cache
def _sc_kernels():
    mesh = plsc.VectorSubcoreMesh(core_axis_name="c", subcore_axis_name="s",
                                  num_cores=NC, num_subcores=NS)
    params = pltpu.CompilerParams(use_tc_tiling_on_sc=False,
                                  needs_layout_passes=False)
    seg = pl.kernel(
        _sc_segment_sum_body,
        out_type=jax.ShapeDtypeStruct((NC, NPAD, S), jnp.float32),
        mesh=mesh,
        scratch_types=[
            pltpu.VMEM((C,), jnp.int32),
            pltpu.VMEM((C,), jnp.int32),
            pltpu.VMEM((C, S), jnp.float32),
            pltpu.VMEM_SHARED((NPAD, S), jnp.float32),
            pltpu.SemaphoreType.DMA,
        ],
        compiler_params=params,
    )
    edot = pl.kernel(
        _sc_edge_dot_body,
        out_type=jax.ShapeDtypeStruct((EPAD2,), jnp.float32),
        mesh=mesh,
        scratch_types=[
            pltpu.VMEM((C,), jnp.int32),
            pltpu.VMEM((C,), jnp.int32),
            pltpu.VMEM((C, S), jnp.float32),
            pltpu.VMEM((C, S), jnp.float32),
            pltpu.VMEM((C,), jnp.float32),
            pltpu.SemaphoreType.DMA,
            pltpu.SemaphoreType.DMA,
        ],
        compiler_params=params,
    )
    return seg, edot


def _tc_in_body(x_ref, wi_ref, wm_ref, h_ref, hm_ref):
    h = jnp.dot(x_ref[...], wi_ref[...], preferred_element_type=jnp.float32)
    h_ref[...] = h
    hm_ref[...] = jnp.dot(h, wm_ref[...], preferred_element_type=jnp.float32)


_tc_in = pl.pallas_call(
    _tc_in_body,
    grid=(GRID,),
    in_specs=[
        pl.BlockSpec((BLK, D), lambda i: (i, 0)),
        pl.BlockSpec((D, S), lambda i: (0, 0)),
        pl.BlockSpec((S, S), lambda i: (0, 0)),
    ],
    out_specs=[pl.BlockSpec((BLK, S), lambda i: (i, 0)),
               pl.BlockSpec((BLK, S), lambda i: (i, 0))],
    out_shape=[jax.ShapeDtypeStruct((NPAD, S), jnp.float32),
               jax.ShapeDtypeStruct((NPAD, S), jnp.float32)],
)


def _tc_round_body(h_ref, m_ref, wm_ref, h_out, hm_out):
    hn = jnp.maximum(h_ref[...] + m_ref[0] + m_ref[1], 0.0)
    h_out[...] = hn
    hm_out[...] = jnp.dot(hn, wm_ref[...], preferred_element_type=jnp.float32)


_tc_round = pl.pallas_call(
    _tc_round_body,
    grid=(GRID,),
    in_specs=[
        pl.BlockSpec((BLK, S), lambda i: (i, 0)),
        pl.BlockSpec((2, BLK, S), lambda i: (0, i, 0)),
        pl.BlockSpec((S, S), lambda i: (0, 0)),
    ],
    out_specs=[pl.BlockSpec((BLK, S), lambda i: (i, 0)),
               pl.BlockSpec((BLK, S), lambda i: (i, 0))],
    out_shape=[jax.ShapeDtypeStruct((NPAD, S), jnp.float32),
               jax.ShapeDtypeStruct((NPAD, S), jnp.float32)],
)


def _tc_post_body(h_ref, m_ref, eps_ref, wmu_ref, wls_ref, wdec_ref,
                  z_ref, zw2_ref, kl_ref):
    hf = jnp.maximum(h_ref[...] + m_ref[0] + m_ref[1], 0.0)
    mu = jnp.dot(hf, wmu_ref[...], preferred_element_type=jnp.float32)
    ls = jnp.dot(hf, wls_ref[...], preferred_element_type=jnp.float32)
    std = jnp.exp(ls)
    z = mu + eps_ref[...] * std
    z_ref[...] = z
    zw = z * wdec_ref[...][:, 0][None, :]
    zw2_ref[0] = -zw
    zw2_ref[1] = zw

    @pl.when(pl.program_id(0) == 0)
    def _():
        kl_ref[...] = jnp.zeros((1, 1), jnp.float32)

    klb = jnp.sum(0.5 * (mu * mu + std * std - 1.0) - ls)
    kl_ref[...] += jnp.full((1, 1), klb, jnp.float32)


_tc_post = pl.pallas_call(
    _tc_post_body,
    grid=(GRID,),
    in_specs=[
        pl.BlockSpec((BLK, S), lambda i: (i, 0)),
        pl.BlockSpec((2, BLK, S), lambda i: (0, i, 0)),
        pl.BlockSpec((BLK, S), lambda i: (i, 0)),
        pl.BlockSpec((S, S), lambda i: (0, 0)),
        pl.BlockSpec((S, S), lambda i: (0, 0)),
        pl.BlockSpec((S, 1), lambda i: (0, 0)),
    ],
    out_specs=[pl.BlockSpec((BLK, S), lambda i: (i, 0)),
               pl.BlockSpec((2, BLK, S), lambda i: (0, i, 0)),
               pl.BlockSpec((1, 1), lambda i: (0, 0))],
    out_shape=[jax.ShapeDtypeStruct((NPAD, S), jnp.float32),
               jax.ShapeDtypeStruct((2, NPAD, S), jnp.float32),
               jax.ShapeDtypeStruct((1, 1), jnp.float32)],
)


def _tc_final_body(ell_ref, kl_ref, b_ref, out_ref, acc_ref):
    i = pl.program_id(0)
    row = i * FBLK + lax.broadcasted_iota(jnp.int32, (FBLK, C), 0)
    bv = b_ref[0, 0]
    adj = jnp.where(row < POS_ROWS, -bv, bv)
    term = -jax.nn.softplus(ell_ref[...] + adj)
    sblk = jnp.sum(jnp.where(row < VALID_ROWS, term, 0.0))

    @pl.when(i == 0)
    def _():
        acc_ref[0] = 0.0

    acc_ref[0] += sblk

    @pl.when(i == FGRID - 1)
    def _():
        loss = -(acc_ref[0] / TWO_E - kl_ref[0, 0] / N)
        out_ref[...] = jnp.full((1, 1), loss, jnp.float32)


_tc_final = pl.pallas_call(
    _tc_final_body,
    grid=(FGRID,),
    in_specs=[
        pl.BlockSpec((FBLK, C), lambda i: (i, 0)),
        pl.BlockSpec((1, 1), lambda i: (0, 0)),
        pl.BlockSpec((1, 1), lambda i: (0, 0)),
    ],
    out_specs=pl.BlockSpec((1, 1), lambda i: (0, 0)),
    out_shape=jax.ShapeDtypeStruct((1, 1), jnp.float32),
    scratch_shapes=[pltpu.SMEM((1,), jnp.float32)],
)


def kernel(x, edge_index, neg_idx, batch, eps, W_in, W_msg, W_mu, W_logstd,
           W_dec, b_dec):
    f32 = jnp.float32
    src = edge_index[0]
    dst = edge_index[1]
    nu = neg_idx[0]
    nv = neg_idx[1]

    padr = jnp.full((EPAD - E,), N, jnp.int32)
    srcp = jnp.concatenate([src, padr])
    dstp = jnp.concatenate([dst, padr])
    pad2 = jnp.full((EPAD2 - TWO_E,), N, jnp.int32)
    u_idx = jnp.concatenate([src, nu, pad2])
    v_idx = jnp.concatenate([dst, nv + NPAD, pad2])

    x_pad = jnp.zeros((NPAD, D), f32).at[:N].set(x)
    eps_pad = jnp.zeros((NPAD, S), f32).at[:N].set(eps)
    zeros_blk = jnp.zeros((NPAD // NS, S), f32)
    _sc_segment_sum, _sc_edge_dot = _sc_kernels()

    h, hm = _tc_in(x_pad, W_in, W_msg)
    for _ in range(2):
        m2 = _sc_segment_sum(hm, srcp, dstp, zeros_blk)
        h, hm = _tc_round(h, m2, W_msg)
    m2 = _sc_segment_sum(hm, srcp, dstp, zeros_blk)
    z, zw2, kl = _tc_post(h, m2, eps_pad, W_mu, W_logstd, W_dec)
    ell = _sc_edge_dot(z, zw2.reshape(2 * NPAD, S), u_idx, v_idx)
    loss = _tc_final(ell.reshape(FROWS, C), kl, b_dec.reshape(1, 1))
    return loss.reshape(())


# speedup vs baseline: 2.7323x; 1.0666x over previous
import functools

import jax
import jax.numpy as jnp
from jax import lax
from jax.experimental import pallas as pl
from jax.experimental.pallas import tpu as pltpu
from jax.experimental.pallas import tpu_sc as plsc

N = 10000
E = 320000
D = 128
S = 64
TWO_E = 2 * E

NC, NS, L = 2, 16, 16
NW = NC * NS
NPAD = 10240
C = 128
NCH = 80
EPW = NCH * C
EPAD = EPW * NW
NCH2 = 158
EPW2 = NCH2 * C
EPAD2 = EPW2 * NW

BLK = 1024
GRID = NPAD // BLK
FROWS = EPAD2 // C
FGRID = 4
FBLK = FROWS // FGRID
POS_ROWS = E // C
VALID_ROWS = TWO_E // C


def _sc_segment_sum_body(hm_hbm, srcp_hbm, dstp_hbm, zeros_hbm, out_hbm,
                         srcA, dstA, srcB, dstB, rowsA, rowsB, acc_sh,
                         semA, semB):
    cid = lax.axis_index("c")
    sid = lax.axis_index("s")
    wid = sid * NC + cid
    base0 = wid * EPW
    zrows = NPAD // NS
    pltpu.sync_copy(zeros_hbm, acc_sh.at[pl.ds(sid * zrows, zrows)])
    plsc.subcore_barrier()

    def load_idx(c, sbuf, dbuf):
        pltpu.sync_copy(srcp_hbm.at[pl.ds(base0 + c * C, C)], sbuf)
        pltpu.sync_copy(dstp_hbm.at[pl.ds(base0 + c * C, C)], dbuf)

    load_idx(0, srcA, dstA)
    pltpu.async_copy(hm_hbm.at[srcA], rowsA, semA)

    def body(k, carry):
        c0 = 2 * k
        load_idx(c0 + 1, srcB, dstB)
        pltpu.async_copy(hm_hbm.at[srcB], rowsB, semB)
        pltpu.make_async_copy(hm_hbm.at[srcA], rowsA, semA).wait()
        pltpu.sync_copy(rowsA, acc_sh.at[dstA], add=True)
        load_idx(c0 + 2, srcA, dstA)
        pltpu.async_copy(hm_hbm.at[srcA], rowsA, semA)
        pltpu.make_async_copy(hm_hbm.at[srcB], rowsB, semB).wait()
        pltpu.sync_copy(rowsB, acc_sh.at[dstB], add=True)
        return carry

    lax.fori_loop(0, NCH // 2, body, 0)
    pltpu.make_async_copy(hm_hbm.at[srcA], rowsA, semA).wait()
    plsc.subcore_barrier()
    rpw = NPAD // NS
    pltpu.sync_copy(acc_sh.at[pl.ds(sid * rpw, rpw)],
                    out_hbm.at[cid, pl.ds(sid * rpw, rpw)])


def _sc_edge_dot_body(z_hbm, zw2_hbm, u_hbm, v_hbm, out_hbm,
                      uA, vA, uB, vB, ubA, vbA, ubB, vbB, ev,
                      sAu, sAv, sBu, sBv):
    cid = lax.axis_index("c")
    sid = lax.axis_index("s")
    wid = sid * NC + cid
    base0 = wid * EPW2
    iota = lax.iota(jnp.int32, L)
    rows = [iota + g * L for g in range(C // L)]
    zero16 = jnp.zeros((L,), jnp.float32)

    def load_idx(c, ubuf, vbuf):
        pltpu.sync_copy(u_hbm.at[pl.ds(base0 + c * C, C)], ubuf)
        pltpu.sync_copy(v_hbm.at[pl.ds(base0 + c * C, C)], vbuf)

    def start_gather(ubuf, vbuf, ub, vb, su, sv):
        pltpu.async_copy(z_hbm.at[ubuf], ub, su)
        pltpu.async_copy(zw2_hbm.at[vbuf], vb, sv)

    def wait_gather(ubuf, vbuf, ub, vb, su, sv):
        pltpu.make_async_copy(z_hbm.at[ubuf], ub, su).wait()
        pltpu.make_async_copy(zw2_hbm.at[vbuf], vb, sv).wait()

    def compute(ub, vb, c):
        @plsc.parallel_loop(0, S, unroll=8,
                            carry=tuple(zero16 for _ in range(C // L)))
        def accs(s, acc):
            col = jnp.full((L,), 0, jnp.int32) + s
            return tuple(
                acc[g]
                + plsc.load_gather(ub, [rows[g], col])
                * plsc.load_gather(vb, [rows[g], col])
                for g in range(C // L))

        for g in range(C // L):
            ev[pl.ds(g * L, L)] = accs[g]
        pltpu.sync_copy(ev, out_hbm.at[pl.ds(base0 + c * C, C)])

    load_idx(0, uA, vA)
    start_gather(uA, vA, ubA, vbA, sAu, sAv)

    def body(k, carry):
        c0 = 2 * k
        load_idx(c0 + 1, uB, vB)
        start_gather(uB, vB, ubB, vbB, sBu, sBv)
        wait_gather(uA, vA, ubA, vbA, sAu, sAv)
        compute(ubA, vbA, c0)
        load_idx(c0 + 2, uA, vA)
        start_gather(uA, vA, ubA, vbA, sAu, sAv)
        wait_gather(uB, vB, ubB, vbB, sBu, sBv)
        compute(ubB, vbB, c0 + 1)
        return carry

    lax.fori_loop(0, NCH2 // 2, body, 0)
    wait_gather(uA, vA, ubA, vbA, sAu, sAv)


@functools.cache
def _sc_kernels():
    mesh = plsc.VectorSubcoreMesh(core_axis_name="c", subcore_axis_name="s",
                                  num_cores=NC, num_subcores=NS)
    params = pltpu.CompilerParams(use_tc_tiling_on_sc=False,
                                  needs_layout_passes=False)
    seg = pl.kernel(
        _sc_segment_sum_body,
        out_type=jax.ShapeDtypeStruct((NC, NPAD, S), jnp.float32),
        mesh=mesh,
        scratch_types=[
            pltpu.VMEM((C,), jnp.int32),
            pltpu.VMEM((C,), jnp.int32),
            pltpu.VMEM((C,), jnp.int32),
            pltpu.VMEM((C,), jnp.int32),
            pltpu.VMEM((C, S), jnp.float32),
            pltpu.VMEM((C, S), jnp.float32),
            pltpu.VMEM_SHARED((NPAD, S), jnp.float32),
            pltpu.SemaphoreType.DMA,
            pltpu.SemaphoreType.DMA,
        ],
        compiler_params=params,
    )
    edot = pl.kernel(
        _sc_edge_dot_body,
        out_type=jax.ShapeDtypeStruct((EPAD2,), jnp.float32),
        mesh=mesh,
        scratch_types=[
            pltpu.VMEM((C,), jnp.int32),
            pltpu.VMEM((C,), jnp.int32),
            pltpu.VMEM((C,), jnp.int32),
            pltpu.VMEM((C,), jnp.int32),
            pltpu.VMEM((C, S), jnp.float32),
            pltpu.VMEM((C, S), jnp.float32),
            pltpu.VMEM((C, S), jnp.float32),
            pltpu.VMEM((C, S), jnp.float32),
            pltpu.VMEM((C,), jnp.float32),
            pltpu.SemaphoreType.DMA,
            pltpu.SemaphoreType.DMA,
            pltpu.SemaphoreType.DMA,
            pltpu.SemaphoreType.DMA,
        ],
        compiler_params=params,
    )
    return seg, edot


def _tc_in_body(x_ref, wi_ref, wm_ref, h_ref, hm_ref):
    h = jnp.dot(x_ref[...], wi_ref[...], preferred_element_type=jnp.float32)
    h_ref[...] = h
    hm_ref[...] = jnp.dot(h, wm_ref[...], preferred_element_type=jnp.float32)


_tc_in = pl.pallas_call(
    _tc_in_body,
    grid=(GRID,),
    in_specs=[
        pl.BlockSpec((BLK, D), lambda i: (i, 0)),
        pl.BlockSpec((D, S), lambda i: (0, 0)),
        pl.BlockSpec((S, S), lambda i: (0, 0)),
    ],
    out_specs=[pl.BlockSpec((BLK, S), lambda i: (i, 0)),
               pl.BlockSpec((BLK, S), lambda i: (i, 0))],
    out_shape=[jax.ShapeDtypeStruct((NPAD, S), jnp.float32),
               jax.ShapeDtypeStruct((NPAD, S), jnp.float32)],
)


def _tc_round_body(h_ref, m_ref, wm_ref, h_out, hm_out):
    hn = jnp.maximum(h_ref[...] + m_ref[0] + m_ref[1], 0.0)
    h_out[...] = hn
    hm_out[...] = jnp.dot(hn, wm_ref[...], preferred_element_type=jnp.float32)


_tc_round = pl.pallas_call(
    _tc_round_body,
    grid=(GRID,),
    in_specs=[
        pl.BlockSpec((BLK, S), lambda i: (i, 0)),
        pl.BlockSpec((2, BLK, S), lambda i: (0, i, 0)),
        pl.BlockSpec((S, S), lambda i: (0, 0)),
    ],
    out_specs=[pl.BlockSpec((BLK, S), lambda i: (i, 0)),
               pl.BlockSpec((BLK, S), lambda i: (i, 0))],
    out_shape=[jax.ShapeDtypeStruct((NPAD, S), jnp.float32),
               jax.ShapeDtypeStruct((NPAD, S), jnp.float32)],
)


def _tc_post_body(h_ref, m_ref, eps_ref, wmu_ref, wls_ref, wdec_ref,
                  z_ref, zw2_ref, kl_ref):
    hf = jnp.maximum(h_ref[...] + m_ref[0] + m_ref[1], 0.0)
    mu = jnp.dot(hf, wmu_ref[...], preferred_element_type=jnp.float32)
    ls = jnp.dot(hf, wls_ref[...], preferred_element_type=jnp.float32)
    std = jnp.exp(ls)
    z = mu + eps_ref[...] * std
    z_ref[...] = z
    zw = z * wdec_ref[...][:, 0][None, :]
    zw2_ref[0] = -zw
    zw2_ref[1] = zw

    @pl.when(pl.program_id(0) == 0)
    def _():
        kl_ref[...] = jnp.zeros((1, 1), jnp.float32)

    klb = jnp.sum(0.5 * (mu * mu + std * std - 1.0) - ls)
    kl_ref[...] += jnp.full((1, 1), klb, jnp.float32)


_tc_post = pl.pallas_call(
    _tc_post_body,
    grid=(GRID,),
    in_specs=[
        pl.BlockSpec((BLK, S), lambda i: (i, 0)),
        pl.BlockSpec((2, BLK, S), lambda i: (0, i, 0)),
        pl.BlockSpec((BLK, S), lambda i: (i, 0)),
        pl.BlockSpec((S, S), lambda i: (0, 0)),
        pl.BlockSpec((S, S), lambda i: (0, 0)),
        pl.BlockSpec((S, 1), lambda i: (0, 0)),
    ],
    out_specs=[pl.BlockSpec((BLK, S), lambda i: (i, 0)),
               pl.BlockSpec((2, BLK, S), lambda i: (0, i, 0)),
               pl.BlockSpec((1, 1), lambda i: (0, 0))],
    out_shape=[jax.ShapeDtypeStruct((NPAD, S), jnp.float32),
               jax.ShapeDtypeStruct((2, NPAD, S), jnp.float32),
               jax.ShapeDtypeStruct((1, 1), jnp.float32)],
)


def _tc_final_body(ell_ref, kl_ref, b_ref, out_ref, acc_ref):
    i = pl.program_id(0)
    row = i * FBLK + lax.broadcasted_iota(jnp.int32, (FBLK, C), 0)
    bv = b_ref[0, 0]
    adj = jnp.where(row < POS_ROWS, -bv, bv)
    term = -jax.nn.softplus(ell_ref[...] + adj)
    sblk = jnp.sum(jnp.where(row < VALID_ROWS, term, 0.0))

    @pl.when(i == 0)
    def _():
        acc_ref[0] = 0.0

    acc_ref[0] += sblk

    @pl.when(i == FGRID - 1)
    def _():
        loss = -(acc_ref[0] / TWO_E - kl_ref[0, 0] / N)
        out_ref[...] = jnp.full((1, 1), loss, jnp.float32)


_tc_final = pl.pallas_call(
    _tc_final_body,
    grid=(FGRID,),
    in_specs=[
        pl.BlockSpec((FBLK, C), lambda i: (i, 0)),
        pl.BlockSpec((1, 1), lambda i: (0, 0)),
        pl.BlockSpec((1, 1), lambda i: (0, 0)),
    ],
    out_specs=pl.BlockSpec((1, 1), lambda i: (0, 0)),
    out_shape=jax.ShapeDtypeStruct((1, 1), jnp.float32),
    scratch_shapes=[pltpu.SMEM((1,), jnp.float32)],
)


def kernel(x, edge_index, neg_idx, batch, eps, W_in, W_msg, W_mu, W_logstd,
           W_dec, b_dec):
    f32 = jnp.float32
    src = edge_index[0]
    dst = edge_index[1]
    nu = neg_idx[0]
    nv = neg_idx[1]

    padr = jnp.full((EPAD + C - E,), N, jnp.int32)
    srcp = jnp.concatenate([src, padr])
    dstp = jnp.concatenate([dst, padr])
    pad2 = jnp.full((EPAD2 + C - TWO_E,), N, jnp.int32)
    u_idx = jnp.concatenate([src, nu, pad2])
    v_idx = jnp.concatenate([dst, nv + NPAD, pad2])

    x_pad = jnp.zeros((NPAD, D), f32).at[:N].set(x)
    eps_pad = jnp.zeros((NPAD, S), f32).at[:N].set(eps)
    zeros_blk = jnp.zeros((NPAD // NS, S), f32)
    _sc_segment_sum, _sc_edge_dot = _sc_kernels()

    h, hm = _tc_in(x_pad, W_in, W_msg)
    for _ in range(2):
        m2 = _sc_segment_sum(hm, srcp, dstp, zeros_blk)
        h, hm = _tc_round(h, m2, W_msg)
    m2 = _sc_segment_sum(hm, srcp, dstp, zeros_blk)
    z, zw2, kl = _tc_post(h, m2, eps_pad, W_mu, W_logstd, W_dec)
    ell = _sc_edge_dot(z, zw2.reshape(2 * NPAD, S), u_idx, v_idx)
    loss = _tc_final(ell.reshape(FROWS, C), kl, b_dec.reshape(1, 1))
    return loss.reshape(())
